# Initial kernel scaffold; baseline (speedup 1.0000x reference)
import jax
import jax.numpy as jnp
from jax.experimental import pallas as pl


def kernel(X, transformation):
    raise NotImplementedError("write your pallas kernel here")


# speedup vs baseline: 1.1455x; 1.1455x over previous
import functools

import jax
import jax.numpy as jnp
import numpy as np
from jax import lax
from jax.experimental import pallas as pl
from jax.experimental.pallas import tpu as pltpu
from jax.experimental.pallas import tpu_sc as plsc

B, H, W, C = 4, 224, 224, 192
HW = H * W
NPIX = B * HW
LANES = 16
CH = 112
GROUPS = CH // LANES
CHUNKS_PER_ROW = W // CH
TILES_PER_IMG = 8
ROWS_PER_TILE = H // TILES_PER_IMG
CVECS = C // LANES


def _tec_body(img, xs_hbm, ys_hbm, out, xsv, ysv, idxa, idxb, idxc, idxd,
              wav, wbv, wcv, wdv, bufa, bufb, bufc, bufd, outb, gsem):
    c_id = lax.axis_index("c")
    s_id = lax.axis_index("s")
    wid = s_id * 2 + c_id
    batch = wid // TILES_PER_IMG
    j0 = (wid - batch * TILES_PER_IMG) * ROWS_PER_TILE
    bb = batch * HW

    def row_body(jj, _):
        j = j0 + jj

        def chunk_body(ck, _):
            row_start = bb + j * W + ck * CH
            pltpu.sync_copy(xs_hbm.at[pl.ds(row_start, CH)], xsv)
            pltpu.sync_copy(ys_hbm.at[pl.ds(row_start, CH)], ysv)
            for g in range(GROUPS):
                sl = pl.ds(g * LANES, LANES)
                xs = xsv[sl]
                ys = ysv[sl]
                x0 = xs.astype(jnp.int32)
                y0 = ys.astype(jnp.int32)
                x0c = jnp.clip(x0, 0, H - 1)
                x1c = jnp.clip(x0 + 1, 0, H - 1)
                y0c = jnp.clip(y0, 0, W - 1)
                y1c = jnp.clip(y0 + 1, 0, W - 1)
                x0f = x0c.astype(jnp.float32)
                x1f = x1c.astype(jnp.float32)
                y0f = y0c.astype(jnp.float32)
                y1f = y1c.astype(jnp.float32)
                wxl = x1f - xs
                wxr = xs - x0f
                wyt = y1f - ys
                wyb = ys - y0f
                wav[sl] = wxl * wyt
                wbv[sl] = wxl * wyb
                wcv[sl] = wxr * wyt
                wdv[sl] = wxr * wyb
                idxa[sl] = bb + y0c * W + x0c
                idxb[sl] = bb + y1c * W + x0c
                idxc[sl] = bb + y0c * W + x1c
                idxd[sl] = bb + y1c * W + x1c
            ca = pltpu.async_copy(img.at[idxa], bufa, gsem)
            cb = pltpu.async_copy(img.at[idxb], bufb, gsem)
            cc = pltpu.async_copy(img.at[idxc], bufc, gsem)
            cd = pltpu.async_copy(img.at[idxd], bufd, gsem)
            ca.wait()
            cb.wait()
            cc.wait()
            cd.wait()

            def pt_body(p, _):
                pidx = jnp.full((LANES,), p, jnp.int32)
                wa = plsc.load_gather(wav, [pidx])
                wb = plsc.load_gather(wbv, [pidx])
                wc = plsc.load_gather(wcv, [pidx])
                wd = plsc.load_gather(wdv, [pidx])
                for cv in range(CVECS):
                    sl = pl.ds(cv * LANES, LANES)
                    acc = ((wa * bufa[p, sl] + wb * bufb[p, sl])
                           + wc * bufc[p, sl]) + wd * bufd[p, sl]
                    outb[p, sl] = acc
                return 0

            lax.fori_loop(0, CH, pt_body, 0)
            pltpu.sync_copy(outb, out.at[pl.ds(row_start, CH)])
            return 0

        lax.fori_loop(0, CHUNKS_PER_ROW, chunk_body, 0)
        return 0

    lax.fori_loop(0, ROWS_PER_TILE, row_body, 0)


@jax.jit
def _sc_interp(img, xs_flat, ys_flat):
    mesh = plsc.VectorSubcoreMesh(core_axis_name="c", subcore_axis_name="s")
    fn = pl.kernel(
        _tec_body,
        mesh=mesh,
        compiler_params=pltpu.CompilerParams(
            needs_layout_passes=False, use_tc_tiling_on_sc=False),
        out_type=jax.ShapeDtypeStruct((NPIX, C), jnp.float32),
        scratch_types=[
            pltpu.VMEM((CH,), jnp.float32),
            pltpu.VMEM((CH,), jnp.float32),
            pltpu.VMEM((CH,), jnp.int32),
            pltpu.VMEM((CH,), jnp.int32),
            pltpu.VMEM((CH,), jnp.int32),
            pltpu.VMEM((CH,), jnp.int32),
            pltpu.VMEM((CH,), jnp.float32),
            pltpu.VMEM((CH,), jnp.float32),
            pltpu.VMEM((CH,), jnp.float32),
            pltpu.VMEM((CH,), jnp.float32),
            pltpu.VMEM((CH, C), jnp.float32),
            pltpu.VMEM((CH, C), jnp.float32),
            pltpu.VMEM((CH, C), jnp.float32),
            pltpu.VMEM((CH, C), jnp.float32),
            pltpu.VMEM((CH, C), jnp.float32),
            pltpu.SemaphoreType.DMA,
        ],
    )
    return fn(img, xs_flat, ys_flat)


def kernel(X, transformation):
    x_linspace = jnp.linspace(-1.0, 1.0, W)
    y_linspace = jnp.linspace(-1.0, 1.0, H)
    x_coordinates, y_coordinates = jnp.meshgrid(x_linspace, y_linspace)
    x_coordinates = x_coordinates.reshape(-1)
    y_coordinates = y_coordinates.reshape(-1)
    ones = jnp.ones_like(x_coordinates)
    grid = jnp.concatenate([x_coordinates, y_coordinates, ones], axis=0)
    grids = jnp.tile(grid.reshape(-1), (B,)).reshape(B, 3, HW)
    transformations = transformation.reshape(B, 2, 3)
    sampled_grids = jnp.einsum('bij,bjk->bik', transformations, grids)
    x = sampled_grids[:, 0:1, :].reshape(-1).astype(jnp.float32)
    y = sampled_grids[:, 1:2, :].reshape(-1).astype(jnp.float32)
    x = 0.5 * (x + 1.0) * jnp.float32(H)
    y = 0.5 * (y + 1.0) * jnp.float32(W)

    img = X.reshape(NPIX, C)
    out = _sc_interp(img, x, y)
    return out.reshape(B, H, W, C)
